# Initial kernel scaffold; baseline (speedup 1.0000x reference)
import jax
import jax.numpy as jnp
from jax.experimental import pallas as pl


def kernel(inputs, tables):
    raise NotImplementedError("write your pallas kernel here")


# speedup vs baseline: 1.6807x; 1.6807x over previous
import functools

import jax
import jax.numpy as jnp
from jax import lax
from jax.experimental import pallas as pl
from jax.experimental.pallas import tpu as pltpu
from jax.experimental.pallas import tpu_sc as plsc

VOCAB = 33
HIDDEN = 512
NUM_TABLES = 6
B = 4 * 8192
NC, NS = 2, 16
NW = NC * NS
ROWS_PER_W = B // NW
CHUNK = 128
NCHUNK = ROWS_PER_W // CHUNK


def _make_sc_lookup():
  mesh = plsc.VectorSubcoreMesh(
      core_axis_name="c", subcore_axis_name="s", num_cores=NC, num_subcores=NS
  )
  out_type = [
      jax.ShapeDtypeStruct((B, HIDDEN), jnp.float32) for _ in range(NUM_TABLES)
  ]
  scratch = [
      pltpu.VMEM((CHUNK,), jnp.int32),
      pltpu.VMEM((CHUNK, HIDDEN), jnp.float32),
      pltpu.SemaphoreType.DMA,
  ]

  @functools.partial(
      pl.kernel, mesh=mesh, out_type=out_type, scratch_types=scratch
  )
  def lookup(idx_hbm, t0, t1, t2, t3, t4, t5, o0, o1, o2, o3, o4, o5,
             idx_v, rows_v, sem):
    tables = (t0, t1, t2, t3, t4, t5)
    outs = (o0, o1, o2, o3, o4, o5)
    wid = lax.axis_index("s") * NC + lax.axis_index("c")
    base0 = wid * ROWS_PER_W

    def chunk_body(c, carry):
      base = base0 + c * CHUNK
      pltpu.sync_copy(idx_hbm.at[pl.ds(base, CHUNK)], idx_v)
      for t in range(NUM_TABLES):
        pltpu.async_copy(tables[t].at[idx_v], rows_v, sem).wait()
        pltpu.sync_copy(rows_v, outs[t].at[pl.ds(base, CHUNK)])
      return carry

    lax.fori_loop(0, NCHUNK, chunk_body, 0)

  return lookup


_sc_lookup = _make_sc_lookup()


def kernel(inputs, tables):
  idx = inputs.reshape(-1).astype(jnp.int32)
  tbls = [tables[i] for i in range(NUM_TABLES)]
  flat = _sc_lookup(idx, *tbls)
  ve = [o.reshape(inputs.shape + (HIDDEN,)) for o in flat]
  return tuple(ve + list(reversed(ve)))


# speedup vs baseline: 1.8236x; 1.0850x over previous
import functools

import jax
import jax.numpy as jnp
from jax import lax
from jax.experimental import pallas as pl
from jax.experimental.pallas import tpu as pltpu
from jax.experimental.pallas import tpu_sc as plsc

VOCAB = 33
HIDDEN = 512
NUM_TABLES = 6
B = 4 * 8192
NC, NS = 2, 16
NW = NC * NS
ROWS_PER_W = B // NW
CHUNK = 64
NCHUNK = ROWS_PER_W // CHUNK


def _make_sc_lookup():
  mesh = plsc.VectorSubcoreMesh(
      core_axis_name="c", subcore_axis_name="s", num_cores=NC, num_subcores=NS
  )
  out_type = [
      jax.ShapeDtypeStruct((B, HIDDEN), jnp.float32) for _ in range(NUM_TABLES)
  ]
  scratch = [
      pltpu.VMEM((NCHUNK, CHUNK), jnp.int32),
      pltpu.VMEM((CHUNK, HIDDEN), jnp.float32),
      pltpu.VMEM((CHUNK, HIDDEN), jnp.float32),
      pltpu.SemaphoreType.DMA,
      pltpu.SemaphoreType.DMA,
  ]

  @functools.partial(
      pl.kernel, mesh=mesh, out_type=out_type, scratch_types=scratch
  )
  def lookup(idx_hbm, t0, t1, t2, t3, t4, t5, o0, o1, o2, o3, o4, o5,
             idx_v, rows_a, rows_b, gsem, ssem):
    tables = (t0, t1, t2, t3, t4, t5)
    outs = (o0, o1, o2, o3, o4, o5)
    bufs = (rows_a, rows_b)
    wid = lax.axis_index("s") * NC + lax.axis_index("c")
    base0 = wid * ROWS_PER_W

    pltpu.sync_copy(idx_hbm.at[wid], idx_v)

    def start_gather(t, c_idx, buf):
      pltpu.async_copy(tables[t].at[idx_v.at[c_idx]], buf, gsem)

    def wait_gather(buf):
      pltpu.make_async_copy(t0.at[idx_v.at[0]], buf, gsem).wait()

    def wait_store():
      pltpu.make_async_copy(rows_a, o0.at[pl.ds(0, CHUNK)], ssem).wait()

    start_gather(0, 0, rows_a)

    def chunk_body(c, carry):
      base = base0 + c * CHUNK
      for t in range(NUM_TABLES):
        buf = bufs[t % 2]
        wait_gather(buf)
        pltpu.async_copy(buf, outs[t].at[pl.ds(base, CHUNK)], ssem)
        if t == 0:
          @pl.when(c > 0)
          def _():
            wait_store()
        else:
          wait_store()
        if t < NUM_TABLES - 1:
          start_gather(t + 1, c, bufs[(t + 1) % 2])
        else:
          @pl.when(c < NCHUNK - 1)
          def _():
            start_gather(0, c + 1, bufs[0])
      return carry

    lax.fori_loop(0, NCHUNK, chunk_body, 0)
    wait_store()

  return lookup


_sc_lookup = _make_sc_lookup()


def kernel(inputs, tables):
  idx = inputs.reshape(NW, NCHUNK, CHUNK).astype(jnp.int32)
  tbls = [tables[i] for i in range(NUM_TABLES)]
  flat = _sc_lookup(idx, *tbls)
  ve = [o.reshape(inputs.shape + (HIDDEN,)) for o in flat]
  return tuple(ve + list(reversed(ve)))
